# baseline (speedup 1.0000x reference)
import functools

import jax
import jax.numpy as jnp
from jax import lax
from jax.experimental import pallas as pl
from jax.experimental.pallas import tpu as pltpu
from jax.experimental.pallas import tpu_sc as plsc

N_NODES = 10000
N_EDGES = 320000
IN_FEATS = 128
H_FEATS = 16
NSEG = H_FEATS

NC = 2
NS = 16
NW = NC * NS
CHUNK = 128
NCHUNK = 80
EPT = NCHUNK * CHUNK
E_PAD = NW * EPT
N_PAD_SLOTS = 112
N_TOT = N_NODES + N_PAD_SLOTS
FLAT = NSEG * N_TOT
GA, GB = 8, 8

_MESH = plsc.VectorSubcoreMesh(core_axis_name="c", subcore_axis_name="s")


@functools.partial(
    pl.kernel,
    out_type=[
        jax.ShapeDtypeStruct((NC * N_TOT,), jnp.float32),
        jax.ShapeDtypeStruct((NC * N_TOT,), jnp.float32),
    ],
    mesh=_MESH,
    scratch_types=[
        pltpu.VMEM((NCHUNK, CHUNK), jnp.int32),
        pltpu.VMEM((NCHUNK, CHUNK), jnp.int32),
        pltpu.VMEM((CHUNK,), jnp.float32),
        pltpu.VMEM_SHARED((N_TOT,), jnp.float32),
        pltpu.VMEM_SHARED((N_TOT,), jnp.float32),
        pltpu.SemaphoreType.DMA,
    ],
)
def _sc_degrees(src_hbm, dst_hbm, z1_hbm, ones_hbm, dout_hbm, din_hbm,
                src_v, dst_v, ones_v, dout_s, din_s, sem_d):
    cid = lax.axis_index("c")
    sid = lax.axis_index("s")
    wid = sid * NC + cid

    pltpu.sync_copy(src_hbm.at[wid], src_v)
    pltpu.sync_copy(dst_hbm.at[wid], dst_v)
    pltpu.sync_copy(ones_hbm, ones_v)

    @pl.when(sid == 0)
    def _():
        pltpu.sync_copy(z1_hbm, dout_s)

    @pl.when(sid == 1)
    def _():
        pltpu.sync_copy(z1_hbm, din_s)

    plsc.subcore_barrier()

    def drain_deg(j):
        pltpu.make_async_copy(ones_v, dout_s.at[src_v.at[j]], sem_d).wait()
        pltpu.make_async_copy(ones_v, din_s.at[dst_v.at[j]], sem_d).wait()

    def body(j, carry):
        @pl.when(j > 0)
        def _():
            drain_deg(j)

        pltpu.async_copy(ones_v, dout_s.at[src_v.at[j]], sem_d, add=True)
        pltpu.async_copy(ones_v, din_s.at[dst_v.at[j]], sem_d, add=True)
        return carry

    lax.fori_loop(0, NCHUNK, body, 0)
    drain_deg(NCHUNK - 1)
    plsc.subcore_barrier()

    @pl.when(sid == 0)
    def _():
        pltpu.sync_copy(dout_s, dout_hbm.at[pl.ds(cid * N_TOT, N_TOT)])

    @pl.when(sid == 1)
    def _():
        pltpu.sync_copy(din_s, din_hbm.at[pl.ds(cid * N_TOT, N_TOT)])


@functools.partial(
    pl.kernel,
    out_type=[
        jax.ShapeDtypeStruct((NC * FLAT,), jnp.float32),
    ],
    mesh=_MESH,
    scratch_types=(
        [
            pltpu.VMEM((NCHUNK, CHUNK), jnp.int32),
            pltpu.VMEM((NCHUNK, CHUNK), jnp.int32),
            pltpu.VMEM((GB, CHUNK), jnp.float32),
        ]
        + [pltpu.VMEM_SHARED((N_TOT,), jnp.float32)] * NSEG
        + [pltpu.VMEM_SHARED((N_TOT,), jnp.float32)] * NSEG
        + [pltpu.SemaphoreType.DMA, pltpu.SemaphoreType.DMA]
    ),
)
def _sc_edge_pass(src_hbm, dst_hbm, tab_hbm, z1_hbm, agg_hbm, *refs):
    src_v, dst_v, cols_v = refs[0], refs[1], refs[2]
    tab_s = refs[3:3 + NSEG]
    agg_s = refs[3 + NSEG:3 + 2 * NSEG]
    sem_g, sem_s = refs[3 + 2 * NSEG], refs[4 + 2 * NSEG]
    cid = lax.axis_index("c")
    sid = lax.axis_index("s")
    wid = sid * NC + cid

    pltpu.sync_copy(src_hbm.at[wid], src_v)
    pltpu.sync_copy(dst_hbm.at[wid], dst_v)
    for k in range(NSEG):
        @pl.when(sid == k % NS)
        def _(k=k):
            pltpu.sync_copy(tab_hbm.at[pl.ds(k * N_TOT, N_TOT)], tab_s[k])
            pltpu.sync_copy(z1_hbm, agg_s[k])

    plsc.subcore_barrier()

    def run_group(off, size):
        def drain_scat(j):
            for k in range(size):
                seg = off + k
                sidx = dst_v
                pltpu.make_async_copy(cols_v.at[k],
                                      agg_s[seg].at[sidx.at[j]],
                                      sem_s).wait()

        def body(j, carry):
            gath = []
            for k in range(size):
                seg = off + k
                gidx, sidx = src_v, dst_v

                @pl.when(j > 0)
                def _(k=k, seg=seg, sidx=sidx):
                    pltpu.make_async_copy(cols_v.at[k],
                                          agg_s[seg].at[sidx.at[j]],
                                          sem_s).wait()

                gath.append(pltpu.async_copy(tab_s[seg].at[gidx.at[j]],
                                             cols_v.at[k], sem_g))
            for k in range(size):
                seg = off + k
                sidx = dst_v
                gath[k].wait()
                pltpu.async_copy(cols_v.at[k], agg_s[seg].at[sidx.at[j]],
                                 sem_s, add=True)
            return carry

        lax.fori_loop(0, NCHUNK, body, 0)
        drain_scat(NCHUNK - 1)

    run_group(0, GA)
    run_group(GA, GB)
    plsc.subcore_barrier()

    for k in range(NSEG):
        @pl.when(sid == k % NS)
        def _(k=k):
            pltpu.sync_copy(agg_s[k],
                            agg_hbm.at[pl.ds(cid * FLAT + k * N_TOT, N_TOT)])


@functools.partial(
    pl.kernel,
    out_type=[
        jax.ShapeDtypeStruct((NC * N_TOT,), jnp.float32),
    ],
    mesh=_MESH,
    scratch_types=[
        pltpu.VMEM((NCHUNK, CHUNK), jnp.int32),
        pltpu.VMEM((NCHUNK, CHUNK), jnp.int32),
        pltpu.VMEM((CHUNK,), jnp.float32),
        pltpu.VMEM_SHARED((N_TOT,), jnp.float32),
        pltpu.VMEM_SHARED((N_TOT,), jnp.float32),
        pltpu.SemaphoreType.DMA,
        pltpu.SemaphoreType.DMA,
    ],
)
def _sc_layer2(src_hbm, dst_hbm, c_hbm, z1_hbm, agg2_hbm,
               src_v, dst_v, cbuf_v, c_s, agg2_s, sem_g, sem_s):
    cid = lax.axis_index("c")
    sid = lax.axis_index("s")
    wid = sid * NC + cid

    pltpu.sync_copy(src_hbm.at[wid], src_v)
    pltpu.sync_copy(dst_hbm.at[wid], dst_v)

    @pl.when(sid == 0)
    def _():
        pltpu.sync_copy(c_hbm, c_s)

    @pl.when(sid == 1)
    def _():
        pltpu.sync_copy(z1_hbm, agg2_s)

    plsc.subcore_barrier()

    def drain_scat(j):
        pltpu.make_async_copy(cbuf_v, agg2_s.at[dst_v.at[j]], sem_s).wait()

    def body(j, carry):
        @pl.when(j > 0)
        def _():
            drain_scat(j)

        pltpu.async_copy(c_s.at[src_v.at[j]], cbuf_v, sem_g).wait()
        pltpu.async_copy(cbuf_v, agg2_s.at[dst_v.at[j]], sem_s, add=True)
        return carry

    lax.fori_loop(0, NCHUNK, body, 0)
    drain_scat(NCHUNK - 1)
    plsc.subcore_barrier()

    @pl.when(sid == 0)
    def _():
        pltpu.sync_copy(agg2_s, agg2_hbm.at[pl.ds(cid * N_TOT, N_TOT)])


def _tc_matmul_body(x_ref, w_ref, o_ref):
    o_ref[...] = jnp.dot(x_ref[...], w_ref[...],
                         preferred_element_type=jnp.float32)


def _tc_norms_body(p_ref, do_ref, di_ref, h_ref, nd_ref, ns_ref):
    do = do_ref[0] + do_ref[1]
    di = di_ref[0] + di_ref[1]
    ns = jnp.where(do > 0.0, lax.rsqrt(do), 0.0)
    nd = jnp.where(di > 0.0, lax.rsqrt(di), 0.0)
    h_ref[...] = p_ref[...] * ns[:, None]
    nd_ref[...] = nd
    ns_ref[...] = ns


def _tc_c_body(agg_ref, nd_ref, ns_ref, b1_ref, w2_ref, c_ref):
    aggt = agg_ref[0] + agg_ref[1]
    r = jnp.maximum(aggt * nd_ref[...][None, :] + b1_ref[...], 0.0)
    c = jnp.sum(r * w2_ref[...], axis=0) * ns_ref[...]
    rowid = lax.broadcasted_iota(jnp.int32, (N_TOT,), 0)
    c_ref[...] = jnp.where(rowid < N_NODES, c, 0.0)


def _tc_out_body(agg2_ref, nd_ref, b2_ref, o_ref):
    agg2 = agg2_ref[0] + agg2_ref[1]
    rowid = lax.broadcasted_iota(jnp.int32, (N_TOT,), 0)
    s = jnp.sum(jnp.where(rowid < N_NODES, agg2 * nd_ref[...], 0.0))
    o_ref[...] = s.reshape(1, 1) / N_NODES + b2_ref[...]


def kernel(in_feat, edge_index, W1, b1, W2, b2):
    xp = jnp.pad(in_feat, ((0, N_TOT - N_NODES), (0, 0)))
    pad_idx = N_NODES + (jnp.arange(E_PAD - N_EDGES, dtype=jnp.int32)
                         % N_PAD_SLOTS)
    src_r = jnp.concatenate([edge_index[0], pad_idx]).reshape(NW, NCHUNK, CHUNK)
    dst_r = jnp.concatenate([edge_index[1], pad_idx]).reshape(NW, NCHUNK, CHUNK)
    z1 = jnp.zeros((N_TOT,), jnp.float32)
    ones = jnp.ones((CHUNK,), jnp.float32)

    p = pl.pallas_call(
        _tc_matmul_body,
        out_shape=jax.ShapeDtypeStruct((N_TOT, H_FEATS), jnp.float32),
    )(xp, W1)

    dout_p, din_p = _sc_degrees(src_r, dst_r, z1, ones)

    h1s, nd, ns = pl.pallas_call(
        _tc_norms_body,
        out_shape=[
            jax.ShapeDtypeStruct((N_TOT, H_FEATS), jnp.float32),
            jax.ShapeDtypeStruct((N_TOT,), jnp.float32),
            jax.ShapeDtypeStruct((N_TOT,), jnp.float32),
        ],
    )(p, dout_p.reshape(NC, N_TOT), din_p.reshape(NC, N_TOT))

    tab = h1s.T.reshape(FLAT)

    (agg_p,) = _sc_edge_pass(src_r, dst_r, tab, z1)

    c = pl.pallas_call(
        _tc_c_body,
        out_shape=jax.ShapeDtypeStruct((N_TOT,), jnp.float32),
    )(agg_p.reshape(NC, NSEG, N_TOT), nd, ns,
      b1.reshape(H_FEATS, 1), W2.reshape(H_FEATS, 1))

    (agg2_p,) = _sc_layer2(src_r, dst_r, c, z1)

    out = pl.pallas_call(
        _tc_out_body,
        out_shape=jax.ShapeDtypeStruct((1, 1), jnp.float32),
    )(agg2_p.reshape(NC, N_TOT), nd, b2.reshape(1, 1))
    return out
